# Initial kernel scaffold; baseline (speedup 1.0000x reference)
import jax
import jax.numpy as jnp
from jax.experimental import pallas as pl


def kernel(features, memory_bank):
    raise NotImplementedError("write your pallas kernel here")


# speedup vs baseline: 4.1164x; 4.1164x over previous
import jax
import jax.numpy as jnp
from jax.experimental import pallas as pl
from jax.experimental.pallas import tpu as pltpu

_NQ = 196
_P = 208
_D = 1536
_N = 100000
_T = 2048
_NT = (_N + _T - 1) // _T
_K = 9
_INF = float("inf")


def _body(f_ref, b_ref, o_ref, run_ref):
    i = pl.program_id(0)

    @pl.when(i == 0)
    def _init():
        run_ref[...] = jnp.full((_K, _P, 128), _INF, jnp.float32)

    f = f_ref[...]
    b = b_ref[...]
    fn = jnp.sum(f * f, axis=1, keepdims=True)
    bn = jnp.sum(b * b, axis=1)
    mm = jax.lax.dot_general(
        f.astype(jnp.bfloat16), b.astype(jnp.bfloat16),
        dimension_numbers=(((1,), (1,)), ((), ())),
        preferred_element_type=jnp.float32)
    d2 = (fn - 2.0 * mm) + bn[None, :]
    col = jax.lax.broadcasted_iota(jnp.int32, (_P, _T), 1) + i * _T
    d2 = jnp.where(col < _N, d2, _INF)

    runs = [run_ref[j] for j in range(_K)]
    for c in range(_T // 128):
        cur = d2[:, c * 128:(c + 1) * 128]
        for j in range(_K):
            lo = jnp.minimum(runs[j], cur)
            cur = jnp.maximum(runs[j], cur)
            runs[j] = lo
    for j in range(_K):
        run_ref[j] = runs[j]

    @pl.when(i == pl.num_programs(0) - 1)
    def _fin():
        cand = jnp.concatenate([run_ref[j] for j in range(_K)], axis=1)
        ii = jax.lax.broadcasted_iota(jnp.int32, cand.shape, 1)
        total = jnp.zeros((_P, 1), jnp.float32)
        for _ in range(_K):
            m = jnp.min(cand, axis=1, keepdims=True)
            total = total + jnp.sqrt(jnp.maximum(m, 1e-12))
            hit = cand == m
            first = jnp.min(jnp.where(hit, ii, jnp.int32(1 << 30)),
                            axis=1, keepdims=True)
            cand = jnp.where(ii == first, _INF, cand)
        o_ref[...] = jnp.broadcast_to(total / float(_K), (_P, 128))


def kernel(features, memory_bank):
    f = jnp.pad(features.astype(jnp.float32), ((0, _P - _NQ), (0, 0)))
    out = pl.pallas_call(
        _body,
        grid=(_NT,),
        in_specs=[
            pl.BlockSpec((_P, _D), lambda i: (0, 0)),
            pl.BlockSpec((_T, _D), lambda i: (i, 0)),
        ],
        out_specs=pl.BlockSpec((_P, 128), lambda i: (0, 0)),
        out_shape=jax.ShapeDtypeStruct((_P, 128), jnp.float32),
        scratch_shapes=[pltpu.VMEM((_K, _P, 128), jnp.float32)],
        compiler_params=pltpu.CompilerParams(
            dimension_semantics=("arbitrary",)),
    )(f, memory_bank)
    return out[:_NQ, 0]


# speedup vs baseline: 4.2122x; 1.0233x over previous
import jax
import jax.numpy as jnp
from jax.experimental import pallas as pl
from jax.experimental.pallas import tpu as pltpu

_NQ = 196
_P = 208
_D = 1536
_N = 100000
_T = 2000
_NT = _N // _T
_K = 9
_INF = float("inf")


def _body(f_ref, fn_ref, b_ref, o_ref, run_ref):
    i = pl.program_id(0)

    @pl.when(i == 0)
    def _init():
        run_ref[...] = jnp.full((_K, _P, 128), _INF, jnp.float32)

    f = f_ref[...]
    b = b_ref[...]
    bn = jnp.sum(b * b, axis=1)
    mm = jax.lax.dot_general(
        f, b.astype(jnp.bfloat16),
        dimension_numbers=(((1,), (1,)), ((), ())),
        preferred_element_type=jnp.float32)
    d2 = mm + bn[None, :]

    runs = [run_ref[j] for j in range(_K)]
    nfull = _T // 128
    for c in range(nfull + 1):
        if c < nfull:
            cur = d2[:, c * 128:(c + 1) * 128]
        else:
            rag = d2[:, nfull * 128:_T]
            cur = jnp.concatenate(
                [rag, jnp.full((_P, 128 - (_T - nfull * 128)), _INF,
                               jnp.float32)], axis=1)
        for j in range(_K):
            lo = jnp.minimum(runs[j], cur)
            cur = jnp.maximum(runs[j], cur)
            runs[j] = lo
    for j in range(_K):
        run_ref[j] = runs[j]

    @pl.when(i == pl.num_programs(0) - 1)
    def _fin():
        fn = fn_ref[...]
        cand = jnp.concatenate([run_ref[j] for j in range(_K)], axis=1)
        ii = jax.lax.broadcasted_iota(jnp.int32, cand.shape, 1)
        total = jnp.zeros((_P, 1), jnp.float32)
        for _ in range(_K):
            m = jnp.min(cand, axis=1, keepdims=True)
            total = total + jnp.sqrt(jnp.maximum(m + fn, 1e-12))
            hit = cand == m
            first = jnp.min(jnp.where(hit, ii, jnp.int32(1 << 30)),
                            axis=1, keepdims=True)
            cand = jnp.where(ii == first, _INF, cand)
        o_ref[...] = jnp.broadcast_to(total / float(_K), (_P, 128))


def kernel(features, memory_bank):
    f32 = features.astype(jnp.float32)
    f = jnp.pad(f32, ((0, _P - _NQ), (0, 0)))
    fneg = (-2.0 * f).astype(jnp.bfloat16)
    fn = jnp.sum(f * f, axis=1, keepdims=True)
    out = pl.pallas_call(
        _body,
        grid=(_NT,),
        in_specs=[
            pl.BlockSpec((_P, _D), lambda i: (0, 0)),
            pl.BlockSpec((_P, 1), lambda i: (0, 0)),
            pl.BlockSpec((_T, _D), lambda i: (i, 0)),
        ],
        out_specs=pl.BlockSpec((_P, 128), lambda i: (0, 0)),
        out_shape=jax.ShapeDtypeStruct((_P, 128), jnp.float32),
        scratch_shapes=[pltpu.VMEM((_K, _P, 128), jnp.float32)],
        compiler_params=pltpu.CompilerParams(
            dimension_semantics=("arbitrary",)),
    )(fneg, fn, memory_bank)
    return out[:_NQ, 0]


# speedup vs baseline: 4.4132x; 1.0477x over previous
import jax
import jax.numpy as jnp
from jax.experimental import pallas as pl
from jax.experimental.pallas import tpu as pltpu

_NQ = 196
_P = 208
_D = 1536
_N = 100000
_T = 4000
_NT = _N // _T
_K = 9
_INF = float("inf")


def _body(f_ref, fn_ref, b_ref, o_ref, run_ref):
    i = pl.program_id(0)

    @pl.when(i == 0)
    def _init():
        run_ref[...] = jnp.full((_K, _P, 128), _INF, jnp.float32)

    f = f_ref[...]
    b = b_ref[...]
    bn = jnp.sum(b * b, axis=1)
    mm = jax.lax.dot_general(
        f, b.astype(jnp.bfloat16),
        dimension_numbers=(((1,), (1,)), ((), ())),
        preferred_element_type=jnp.float32)
    d2 = mm + bn[None, :]

    runs = [run_ref[j] for j in range(_K)]
    nfull = _T // 128
    for c in range(nfull + 1):
        if c < nfull:
            cur = d2[:, c * 128:(c + 1) * 128]
        else:
            rag = d2[:, nfull * 128:_T]
            cur = jnp.concatenate(
                [rag, jnp.full((_P, 128 - (_T - nfull * 128)), _INF,
                               jnp.float32)], axis=1)
        for j in range(_K):
            lo = jnp.minimum(runs[j], cur)
            cur = jnp.maximum(runs[j], cur)
            runs[j] = lo
    for j in range(_K):
        run_ref[j] = runs[j]

    @pl.when(i == pl.num_programs(0) - 1)
    def _fin():
        fn = fn_ref[...]
        cand = jnp.concatenate([run_ref[j] for j in range(_K)], axis=1)
        ii = jax.lax.broadcasted_iota(jnp.int32, cand.shape, 1)
        total = jnp.zeros((_P, 1), jnp.float32)
        for _ in range(_K):
            m = jnp.min(cand, axis=1, keepdims=True)
            total = total + jnp.sqrt(jnp.maximum(m + fn, 1e-12))
            hit = cand == m
            first = jnp.min(jnp.where(hit, ii, jnp.int32(1 << 30)),
                            axis=1, keepdims=True)
            cand = jnp.where(ii == first, _INF, cand)
        o_ref[...] = jnp.broadcast_to(total / float(_K), (_P, 128))


def kernel(features, memory_bank):
    f32 = features.astype(jnp.float32)
    f = jnp.pad(f32, ((0, _P - _NQ), (0, 0)))
    fneg = (-2.0 * f).astype(jnp.bfloat16)
    fn = jnp.sum(f * f, axis=1, keepdims=True)
    out = pl.pallas_call(
        _body,
        grid=(_NT,),
        in_specs=[
            pl.BlockSpec((_P, _D), lambda i: (0, 0)),
            pl.BlockSpec((_P, 1), lambda i: (0, 0)),
            pl.BlockSpec((_T, _D), lambda i: (i, 0)),
        ],
        out_specs=pl.BlockSpec((_P, 128), lambda i: (0, 0)),
        out_shape=jax.ShapeDtypeStruct((_P, 128), jnp.float32),
        scratch_shapes=[pltpu.VMEM((_K, _P, 128), jnp.float32)],
        compiler_params=pltpu.CompilerParams(
            dimension_semantics=("arbitrary",)),
    )(fneg, fn, memory_bank)
    return out[:_NQ, 0]
